# baseline (speedup 1.0000x reference)
import functools
import math

import jax
import jax.numpy as jnp
from jax.experimental import pallas as pl
from jax.experimental.pallas import tpu as pltpu


def _emb_kernel(loc_ref, prev_ref, flp_ref, tok_ref, pe_ref, out_ref,
                *, scale, njl):
    j = pl.program_id(0)

    prev = prev_ref[0, 0, 0:1, :]
    flv = flp_ref[0, :, :]
    cls = tok_ref[0:1, :]
    eos = tok_ref[2:3, :]

    @pl.when(j < njl)
    def _bulk():
        lv = loc_ref[0, :, :]
        shifted = pltpu.roll(lv, 1, axis=0)
        base = jnp.where(flv == 3, eos, shifted)
        out_ref[0, :, :] = base * scale + pe_ref[:, :]

    row0_val = jnp.where(j == 0, cls,
                         jnp.where(flv[0:1, :] == 3, eos, prev))
    out_ref[0, 0:1, :] = row0_val * scale + pe_ref[0:1, :]


def kernel(location_embedding, loc_flags, tok_emb, pe):
    b, s, d = location_embedding.shape
    scale = math.sqrt(float(d))

    C = 512
    njl = s // C
    nj = (s + 1 + C - 1) // C

    prevs = location_embedding[:, C - 1::C, :].reshape(b, njl, 1, d)

    flags32 = loc_flags.astype(jnp.int32)
    flp = jnp.concatenate(
        [jnp.zeros((b, 1), jnp.int32), flags32,
         jnp.zeros((b, nj * C - s - 1), jnp.int32)], axis=1
    ).reshape(b, nj * C, 1)

    body = functools.partial(_emb_kernel, scale=scale, njl=njl)

    return pl.pallas_call(
        body,
        grid=(nj, b),
        in_specs=[
            pl.BlockSpec((1, C, d),
                         lambda j, bb: (jnp.where(j < njl, bb, b - 1),
                                        jnp.minimum(j, njl - 1), 0)),
            pl.BlockSpec((1, 1, 1, d),
                         lambda j, bb: (bb, jnp.maximum(j - 1, 0), 0, 0)),
            pl.BlockSpec((1, C, 1), lambda j, bb: (bb, j, 0)),
            pl.BlockSpec(tok_emb.shape, lambda j, bb: (0, 0)),
            pl.BlockSpec((C, d), lambda j, bb: (j, 0)),
        ],
        out_specs=pl.BlockSpec((1, C, d), lambda j, bb: (bb, j, 0)),
        out_shape=jax.ShapeDtypeStruct((b, s + 1, d), jnp.float32),
        compiler_params=pltpu.CompilerParams(
            dimension_semantics=("parallel", "parallel")),
    )(location_embedding, prevs, flp, tok_emb, pe)


# speedup vs baseline: 1.0662x; 1.0662x over previous
import functools
import math

import jax
import jax.numpy as jnp
from jax.experimental import pallas as pl
from jax.experimental.pallas import tpu as pltpu


def _emb_kernel(loc_ref, prev_ref, flp_ref, tok_ref, pe_ref, out_ref,
                *, scale, njl):
    j = pl.program_id(0)

    prev = prev_ref[0, 0, 0:1, :]
    flv = flp_ref[0, :, :]
    cls = tok_ref[0:1, :]
    eos = tok_ref[2:3, :]

    @pl.when(j < njl)
    def _bulk():
        lv = loc_ref[0, :, :]
        shifted = pltpu.roll(lv, 1, axis=0)
        base = jnp.where(flv == 3, eos, shifted)
        out_ref[0, :, :] = base * scale + pe_ref[:, :]

    row0_val = jnp.where(j == 0, cls,
                         jnp.where(flv[0:1, :] == 3, eos, prev))
    out_ref[0, 0:1, :] = row0_val * scale + pe_ref[0:1, :]


def kernel(location_embedding, loc_flags, tok_emb, pe):
    b, s, d = location_embedding.shape
    scale = math.sqrt(float(d))

    C = 2048
    njl = s // C
    nj = (s + 1 + C - 1) // C

    prevs = location_embedding[:, C - 1::C, :].reshape(b, njl, 1, d)

    flags32 = loc_flags.astype(jnp.int32)
    flp = jnp.concatenate(
        [jnp.zeros((b, 1), jnp.int32), flags32,
         jnp.zeros((b, nj * C - s - 1), jnp.int32)], axis=1
    ).reshape(b, nj * C, 1)

    body = functools.partial(_emb_kernel, scale=scale, njl=njl)

    return pl.pallas_call(
        body,
        grid=(nj, b),
        in_specs=[
            pl.BlockSpec((1, C, d),
                         lambda j, bb: (jnp.where(j < njl, bb, b - 1),
                                        jnp.minimum(j, njl - 1), 0)),
            pl.BlockSpec((1, 1, 1, d),
                         lambda j, bb: (bb, jnp.maximum(j - 1, 0), 0, 0)),
            pl.BlockSpec((1, C, 1), lambda j, bb: (bb, j, 0)),
            pl.BlockSpec(tok_emb.shape, lambda j, bb: (0, 0)),
            pl.BlockSpec((C, d), lambda j, bb: (j, 0)),
        ],
        out_specs=pl.BlockSpec((1, C, d), lambda j, bb: (bb, j, 0)),
        out_shape=jax.ShapeDtypeStruct((b, s + 1, d), jnp.float32),
    )(location_embedding, prevs, flp, tok_emb, pe)
